# Initial kernel scaffold; baseline (speedup 1.0000x reference)
import jax
import jax.numpy as jnp
from jax.experimental import pallas as pl


def kernel(z, embedding):
    raise NotImplementedError("write your pallas kernel here")


# speedup vs baseline: 1.2601x; 1.2601x over previous
import jax
import jax.numpy as jnp
from jax.experimental import pallas as pl
from jax.experimental.pallas import tpu as pltpu
from jax.experimental.pallas import tpu_sc as plsc

_B = 32768
_K = 8192
_D = 256
_CC = 0.25

_BB = 256
_NB = _B // _BB
_EB = 2048
_NE = _B // _EB
_GW = 128


def _argmin_body(z_ref, e_ref, idx_ref, esq_ref):
    b = pl.program_id(0)

    @pl.when(b == 0)
    def _():
        e = e_ref[...]
        esq_ref[...] = jnp.sum(e * e, axis=1)[None, :]

    z = z_ref[...]
    zsq = jnp.sum(z * z, axis=1, keepdims=True)
    mm = jax.lax.dot_general(z, e_ref[...], (((1,), (1,)), ((), ())),
                             preferred_element_type=jnp.float32)
    dist = (zsq + esq_ref[...]) - 2.0 * mm
    lmin = jnp.min(dist, axis=1, keepdims=True)
    iota = jax.lax.broadcasted_iota(jnp.int32, dist.shape, 1)
    idx_ref[0, 0, :] = jnp.min(jnp.where(dist == lmin, iota, _K), axis=1)


def _encode(z, embedding):
    idx3 = pl.pallas_call(
        _argmin_body,
        grid=(_NB,),
        in_specs=[
            pl.BlockSpec((_BB, _D), lambda b: (b, 0)),
            pl.BlockSpec((_K, _D), lambda b: (0, 0)),
        ],
        out_specs=pl.BlockSpec((1, 1, _BB), lambda b: (b, 0, 0)),
        out_shape=jax.ShapeDtypeStruct((_NB, 1, _BB), jnp.int32),
        scratch_shapes=[pltpu.VMEM((1, _K), jnp.float32)],
    )(z, embedding)
    return idx3.reshape(_B)


def _gather_rows(embedding, indices):
    idx2 = indices.reshape(1, _B)

    @pl.kernel(
        out_type=jax.ShapeDtypeStruct((_B, _D), jnp.float32),
        mesh=plsc.VectorSubcoreMesh(core_axis_name="c", subcore_axis_name="s"),
    )
    def k(emb_hbm, i_hbm, o_hbm):
        def body(i_vmem, o_vmem):
            pltpu.sync_copy(emb_hbm.at[i_vmem.at[0]], o_vmem)

        pltpu.emit_pipeline(
            body,
            grid=(_B // _GW,),
            in_specs=[pl.BlockSpec((1, _GW), lambda i: (0, i))],
            out_specs=[pl.BlockSpec((_GW, _D), lambda i: (i, 0))],
            core_axis_name=("c", "s"),
            dimension_semantics=(pltpu.PARALLEL,),
        )(i_hbm, o_hbm)

    return k(embedding, idx2)


def _st_loss_body(z_ref, zq_ref, zst_ref, loss_ref, acc_ref):
    b = pl.program_id(0)

    @pl.when(b == 0)
    def _():
        acc_ref[0, 0] = 0.0

    z = z_ref[...]
    d = zq_ref[...] - z
    zst_ref[...] = z + d
    acc_ref[0, 0] += jnp.sum(d * d)

    @pl.when(b == _NE - 1)
    def _():
        m = acc_ref[0, 0] / (_B * _D)
        loss_ref[0, 0] = m + _CC * m


def _st_and_loss(z, z_q):
    z_q_st, loss = pl.pallas_call(
        _st_loss_body,
        grid=(_NE,),
        in_specs=[
            pl.BlockSpec((_EB, _D), lambda b: (b, 0)),
            pl.BlockSpec((_EB, _D), lambda b: (b, 0)),
        ],
        out_specs=[
            pl.BlockSpec((_EB, _D), lambda b: (b, 0)),
            pl.BlockSpec(memory_space=pltpu.SMEM),
        ],
        out_shape=[
            jax.ShapeDtypeStruct((_B, _D), jnp.float32),
            jax.ShapeDtypeStruct((1, 1), jnp.float32),
        ],
        scratch_shapes=[pltpu.SMEM((1, 1), jnp.float32)],
    )(z, z_q)
    return z_q_st, loss.reshape(())


def kernel(z, embedding):
    indices = _encode(z, embedding)
    z_q = _gather_rows(embedding, indices)
    z_q_st, loss = _st_and_loss(z, z_q)
    return (z_q_st, loss, indices)
